# baseline (speedup 1.0000x reference)
import functools

import jax
import jax.numpy as jnp
from jax import lax
from jax.experimental import pallas as pl
from jax.experimental.pallas import tpu as pltpu
from jax.experimental.pallas import tpu_sc as plsc

VOCAB = 1000000
SEQ = 200
DIM = 64
BATCH = 4096
NROWS = BATCH * SEQ
NC, NS, LANES = 2, 16, 16
NW = NC * NS

FC = 512
NFC = VOCAB // FC
FREM = VOCAB - NFC * FC
TAIL0 = NFC * FC

BPW = NROWS // NW
C = 800
NIT = BPW // C
GC = 80
NG = C // GC


def _fmt_body(tokt_hbm, tail_hbm, out_hbm, inbuf, outbuf, tailbuf):
    wid = lax.axis_index("s") * NC + lax.axis_index("c")
    d16 = [lax.iota(jnp.int32, LANES) + j * LANES for j in range(DIM // LANES)]

    def transpose_chunk(width, t, carry):
        tvec = jnp.full((LANES,), t, dtype=jnp.int32)
        for j in range(DIM // LANES):
            v = plsc.load_gather(inbuf, [d16[j], tvec])
            outbuf[pl.ds(t * DIM + j * LANES, LANES)] = v
        return carry

    def chunk_body(tloop, carry):
        cid = wid + NW * tloop

        @pl.when(cid < NFC)
        def _():
            c0 = cid * FC
            pltpu.sync_copy(tokt_hbm.at[:, pl.ds(c0, FC)], inbuf)
            lax.fori_loop(0, FC, functools.partial(transpose_chunk, FC), 0)
            pltpu.sync_copy(outbuf, out_hbm.at[pl.ds(c0 * DIM, FC * DIM)])

        return carry

    lax.fori_loop(0, NFC // NW + 1, chunk_body, 0)

    @pl.when(wid == 0)
    def _():
        pltpu.sync_copy(tail_hbm, tailbuf)
        pltpu.sync_copy(tailbuf, out_hbm.at[pl.ds(TAIL0 * DIM, FREM * DIM)])


def _emb_body(idx_hbm, tok_hbm, pos_hbm, out_hbm, idx_v, rows_v, pos_v, sem):
    wid = lax.axis_index("s") * NC + lax.axis_index("c")
    base = wid * BPW
    pltpu.sync_copy(pos_hbm, pos_v)

    def chunk_body(i, carry):
        off = base + i * C
        pltpu.sync_copy(idx_hbm.at[pl.ds(off, C)], idx_v)
        copies = [
            pltpu.async_copy(
                tok_hbm.at[idx_v.at[pl.ds(g * GC, GC)]],
                rows_v.at[pl.ds(g * GC, GC)],
                sem,
            )
            for g in range(NG)
        ]
        for cp in copies:
            cp.wait()

        def add_body(r, carry2):
            for rep in range(C // SEQ):
                for j in range(DIM // LANES):
                    sl = pl.ds(j * LANES, LANES)
                    rows_v[rep * SEQ + r, sl] = (
                        rows_v[rep * SEQ + r, sl] + pos_v[r, sl]
                    )
            return carry2

        lax.fori_loop(0, SEQ, add_body, 0, unroll=2)
        pltpu.sync_copy(rows_v, out_hbm.at[pl.ds(off, C)])
        return carry

    lax.fori_loop(0, NIT, chunk_body, 0)


@functools.partial(jax.jit, static_argnames=())
def kernel(inputs, token_table, pos_table):
    idx = inputs.reshape(-1).astype(jnp.int32)
    mesh = plsc.VectorSubcoreMesh(core_axis_name="c", subcore_axis_name="s")

    fmt = pl.kernel(
        _fmt_body,
        out_type=jax.ShapeDtypeStruct((VOCAB * DIM,), jnp.float32),
        mesh=mesh,
        scratch_types=[
            pltpu.VMEM((DIM, FC), jnp.float32),
            pltpu.VMEM((FC * DIM,), jnp.float32),
            pltpu.VMEM((FREM * DIM,), jnp.float32),
        ],
        compiler_params=pltpu.CompilerParams(use_tc_tiling_on_sc=True,
                                             needs_layout_passes=False),
    )
    tail_flat = token_table[TAIL0:].reshape(-1)
    packed = fmt(token_table.T, tail_flat)
    tok_lin = packed.reshape(VOCAB, DIM)

    run = pl.kernel(
        _emb_body,
        out_type=jax.ShapeDtypeStruct((NROWS, DIM), jnp.float32),
        mesh=mesh,
        scratch_types=[
            pltpu.VMEM((C,), jnp.int32),
            pltpu.VMEM((C, DIM), jnp.float32),
            pltpu.VMEM((SEQ, DIM), jnp.float32),
            pltpu.SemaphoreType.DMA,
        ],
        compiler_params=pltpu.CompilerParams(use_tc_tiling_on_sc=False),
    )
    out = run(idx, tok_lin, pos_table)
    return out.reshape(BATCH, SEQ, DIM)


# speedup vs baseline: 1.0668x; 1.0668x over previous
import functools

import jax
import jax.numpy as jnp
from jax import lax
from jax.experimental import pallas as pl
from jax.experimental.pallas import tpu as pltpu
from jax.experimental.pallas import tpu_sc as plsc

VOCAB = 1000000
SEQ = 200
DIM = 64
BATCH = 4096
NROWS = BATCH * SEQ
NC, NS, LANES = 2, 16, 16
NW = NC * NS

FC = 128
NFC = VOCAB // FC
FREM = VOCAB - NFC * FC
TAIL0 = NFC * FC

BPW = NROWS // NW
C = 800
NIT = BPW // C
GC = 80
NG = C // GC


def _fmt_body(tokt_hbm, tail_hbm, out_hbm, inbuf, outbuf, tailbuf):
    wid = lax.axis_index("s") * NC + lax.axis_index("c")
    iota64 = lax.iota(jnp.int32, LANES) * DIM

    def chunk_body(tloop, carry):
        cid = wid + NW * tloop

        @pl.when(cid < NFC)
        def _():
            c0 = pl.multiple_of(cid * FC, FC)
            pltpu.sync_copy(tokt_hbm.at[:, pl.ds(c0, FC)], inbuf)
            for d in range(DIM):
                for m in range(FC // LANES):
                    v = inbuf[d, pl.ds(m * LANES, LANES)]
                    plsc.store_scatter(
                        outbuf, [iota64 + (m * LANES * DIM + d)], v)
            pltpu.sync_copy(outbuf, out_hbm.at[pl.ds(c0 * DIM, FC * DIM)])

        return carry

    lax.fori_loop(0, NFC // NW + 1, chunk_body, 0)

    @pl.when(wid == 0)
    def _():
        pltpu.sync_copy(tail_hbm, tailbuf)
        pltpu.sync_copy(tailbuf, out_hbm.at[pl.ds(TAIL0 * DIM, FREM * DIM)])


def _emb_body(idx_hbm, tok_hbm, pos_hbm, out_hbm, idx_v, rows_v, pos_v, sem):
    wid = lax.axis_index("s") * NC + lax.axis_index("c")
    base = wid * BPW
    pltpu.sync_copy(pos_hbm, pos_v)

    def chunk_body(i, carry):
        off = base + i * C
        pltpu.sync_copy(idx_hbm.at[pl.ds(off, C)], idx_v)
        copies = [
            pltpu.async_copy(
                tok_hbm.at[idx_v.at[pl.ds(g * GC, GC)]],
                rows_v.at[pl.ds(g * GC, GC)],
                sem,
            )
            for g in range(NG)
        ]
        for cp in copies:
            cp.wait()

        def add_body(r, carry2):
            for rep in range(C // SEQ):
                for j in range(DIM // LANES):
                    sl = pl.ds(j * LANES, LANES)
                    rows_v[rep * SEQ + r, sl] = (
                        rows_v[rep * SEQ + r, sl] + pos_v[r, sl]
                    )
            return carry2

        lax.fori_loop(0, SEQ, add_body, 0, unroll=2)
        pltpu.sync_copy(rows_v, out_hbm.at[pl.ds(off, C)])
        return carry

    lax.fori_loop(0, NIT, chunk_body, 0)


@functools.partial(jax.jit, static_argnames=())
def kernel(inputs, token_table, pos_table):
    idx = inputs.reshape(-1).astype(jnp.int32)
    mesh = plsc.VectorSubcoreMesh(core_axis_name="c", subcore_axis_name="s")

    fmt = pl.kernel(
        _fmt_body,
        out_type=jax.ShapeDtypeStruct((VOCAB * DIM,), jnp.float32),
        mesh=mesh,
        scratch_types=[
            pltpu.VMEM((DIM, FC), jnp.float32),
            pltpu.VMEM((FC * DIM,), jnp.float32),
            pltpu.VMEM((FREM * DIM,), jnp.float32),
        ],
        compiler_params=pltpu.CompilerParams(use_tc_tiling_on_sc=True,
                                             needs_layout_passes=False),
    )
    tail_flat = token_table[TAIL0:].reshape(-1)
    packed = fmt(token_table.T, tail_flat)
    tok_lin = packed.reshape(VOCAB, DIM)

    run = pl.kernel(
        _emb_body,
        out_type=jax.ShapeDtypeStruct((NROWS, DIM), jnp.float32),
        mesh=mesh,
        scratch_types=[
            pltpu.VMEM((C,), jnp.int32),
            pltpu.VMEM((C, DIM), jnp.float32),
            pltpu.VMEM((SEQ, DIM), jnp.float32),
            pltpu.SemaphoreType.DMA,
        ],
        compiler_params=pltpu.CompilerParams(use_tc_tiling_on_sc=False),
    )
    out = run(idx, tok_lin, pos_table)
    return out.reshape(BATCH, SEQ, DIM)


# speedup vs baseline: 1.2481x; 1.1699x over previous
import functools

import jax
import jax.numpy as jnp
from jax import lax
from jax.experimental import pallas as pl
from jax.experimental.pallas import tpu as pltpu
from jax.experimental.pallas import tpu_sc as plsc

VOCAB = 1000000
SEQ = 200
DIM = 64
BATCH = 4096
NROWS = BATCH * SEQ
NC, NS, LANES = 2, 16, 16
NW = NC * NS

FC = 128
NFC = VOCAB // FC
FREM = VOCAB - NFC * FC
TAIL0 = NFC * FC

BPW = NROWS // NW
C = 800
NIT = BPW // C
GC = 80
NG = C // GC


def _fmt_body(tokt_hbm, tail_hbm, out_hbm, inbuf0, inbuf1, outbuf0,
              outbuf1, tailbuf, sin0, sin1, sout0, sout1):
    wid = lax.axis_index("s") * NC + lax.axis_index("c")
    iota64 = lax.iota(jnp.int32, LANES) * DIM
    inbufs, outbufs = (inbuf0, inbuf1), (outbuf0, outbuf1)
    sins, souts = (sin0, sin1), (sout0, sout1)
    NT = 2 * (NFC // NW // 2 + 1)

    def start_in(t, b):
        cid = wid + NW * t

        @pl.when(cid < NFC)
        def _():
            c0 = pl.multiple_of(cid * FC, FC)
            pltpu.async_copy(tokt_hbm.at[:, pl.ds(c0, FC)], inbufs[b],
                             sins[b])

    def transpose(b):
        for d in range(DIM):
            for m in range(FC // LANES):
                v = inbufs[b][d, pl.ds(m * LANES, LANES)]
                plsc.store_scatter(
                    outbufs[b], [iota64 + (m * LANES * DIM + d)], v)

    start_in(0, 0)
    start_in(1, 1)

    def pair_body(tt, carry):
        for b in range(2):
            t = 2 * tt + b
            cid = wid + NW * t

            @pl.when((t >= 2) & (cid - 2 * NW < NFC))
            def _():
                pltpu.make_async_copy(
                    outbufs[b], out_hbm.at[pl.ds(0, FC * DIM)],
                    souts[b]).wait()

            @pl.when(cid < NFC)
            def _():
                pltpu.make_async_copy(
                    tokt_hbm.at[:, pl.ds(0, FC)], inbufs[b], sins[b]).wait()
                transpose(b)
                start_in(t + 2, b)
                c0 = pl.multiple_of(cid * FC, FC)
                pltpu.async_copy(outbufs[b],
                                 out_hbm.at[pl.ds(c0 * DIM, FC * DIM)],
                                 souts[b])

        return carry

    lax.fori_loop(0, NT // 2, pair_body, 0)

    for b in range(2):
        cid = wid + NW * (NT - 2 + b)

        @pl.when(cid < NFC)
        def _():
            pltpu.make_async_copy(
                outbufs[b], out_hbm.at[pl.ds(0, FC * DIM)], souts[b]).wait()

    @pl.when(wid == 0)
    def _():
        pltpu.sync_copy(tail_hbm, tailbuf)
        pltpu.sync_copy(tailbuf, out_hbm.at[pl.ds(TAIL0 * DIM, FREM * DIM)])


def _emb_body(idx_hbm, tok_hbm, pos_hbm, out_hbm, idx_v, rows_v, pos_v, sem):
    wid = lax.axis_index("s") * NC + lax.axis_index("c")
    base = wid * BPW
    pltpu.sync_copy(pos_hbm, pos_v)

    def chunk_body(i, carry):
        off = base + i * C
        pltpu.sync_copy(idx_hbm.at[pl.ds(off, C)], idx_v)
        copies = [
            pltpu.async_copy(
                tok_hbm.at[idx_v.at[pl.ds(g * GC, GC)]],
                rows_v.at[pl.ds(g * GC, GC)],
                sem,
            )
            for g in range(NG)
        ]
        for cp in copies:
            cp.wait()

        def add_body(r, carry2):
            for rep in range(C // SEQ):
                for j in range(DIM // LANES):
                    sl = pl.ds(j * LANES, LANES)
                    rows_v[rep * SEQ + r, sl] = (
                        rows_v[rep * SEQ + r, sl] + pos_v[r, sl]
                    )
            return carry2

        lax.fori_loop(0, SEQ, add_body, 0, unroll=2)
        pltpu.sync_copy(rows_v, out_hbm.at[pl.ds(off, C)])
        return carry

    lax.fori_loop(0, NIT, chunk_body, 0)


@functools.partial(jax.jit, static_argnames=())
def kernel(inputs, token_table, pos_table):
    idx = inputs.reshape(-1).astype(jnp.int32)
    mesh = plsc.VectorSubcoreMesh(core_axis_name="c", subcore_axis_name="s")

    fmt = pl.kernel(
        _fmt_body,
        out_type=jax.ShapeDtypeStruct((VOCAB * DIM,), jnp.float32),
        mesh=mesh,
        scratch_types=[
            pltpu.VMEM((DIM, FC), jnp.float32),
            pltpu.VMEM((DIM, FC), jnp.float32),
            pltpu.VMEM((FC * DIM,), jnp.float32),
            pltpu.VMEM((FC * DIM,), jnp.float32),
            pltpu.VMEM((FREM * DIM,), jnp.float32),
            pltpu.SemaphoreType.DMA,
            pltpu.SemaphoreType.DMA,
            pltpu.SemaphoreType.DMA,
            pltpu.SemaphoreType.DMA,
        ],
        compiler_params=pltpu.CompilerParams(use_tc_tiling_on_sc=True,
                                             needs_layout_passes=False),
    )
    tail_flat = token_table[TAIL0:].reshape(-1)
    packed = fmt(token_table.T, tail_flat)
    tok_lin = packed.reshape(VOCAB, DIM)

    run = pl.kernel(
        _emb_body,
        out_type=jax.ShapeDtypeStruct((NROWS, DIM), jnp.float32),
        mesh=mesh,
        scratch_types=[
            pltpu.VMEM((C,), jnp.int32),
            pltpu.VMEM((C, DIM), jnp.float32),
            pltpu.VMEM((SEQ, DIM), jnp.float32),
            pltpu.SemaphoreType.DMA,
        ],
        compiler_params=pltpu.CompilerParams(use_tc_tiling_on_sc=False),
    )
    out = run(idx, tok_lin, pos_table)
    return out.reshape(BATCH, SEQ, DIM)


# speedup vs baseline: 1.8132x; 1.4528x over previous
import functools

import jax
import jax.numpy as jnp
from jax import lax
from jax.experimental import pallas as pl
from jax.experimental.pallas import tpu as pltpu
from jax.experimental.pallas import tpu_sc as plsc

VOCAB = 1000000
SEQ = 200
DIM = 64
BATCH = 4096
NROWS = BATCH * SEQ
NC, NS, LANES = 2, 16, 16
NW = NC * NS
BPW = NROWS // NW
C = 800
NIT = BPW // C
GC = 80
NG = C // GC


def _emb_body(idx_hbm, tok_hbm, pos_hbm, out_hbm, idx_v, rows_v, pos_v, sem):
    wid = lax.axis_index("s") * NC + lax.axis_index("c")
    base = wid * BPW
    pltpu.sync_copy(pos_hbm, pos_v)

    def chunk_body(i, carry):
        off = base + i * C
        pltpu.sync_copy(idx_hbm.at[pl.ds(off, C)], idx_v)
        copies = [
            pltpu.async_copy(
                tok_hbm.at[idx_v.at[pl.ds(g * GC, GC)]],
                rows_v.at[pl.ds(g * GC, GC)],
                sem,
            )
            for g in range(NG)
        ]
        for cp in copies:
            cp.wait()

        def add_body(r, carry2):
            for rep in range(C // SEQ):
                for j in range(DIM // LANES):
                    sl = pl.ds(j * LANES, LANES)
                    rows_v[rep * SEQ + r, sl] = (
                        rows_v[rep * SEQ + r, sl] + pos_v[r, sl]
                    )
            return carry2

        lax.fori_loop(0, SEQ, add_body, 0, unroll=2)
        pltpu.sync_copy(rows_v, out_hbm.at[pl.ds(off, C)])
        return carry

    lax.fori_loop(0, NIT, chunk_body, 0)


@functools.partial(jax.jit, static_argnames=())
def kernel(inputs, token_table, pos_table):
    idx = inputs.reshape(-1).astype(jnp.int32)
    mesh = plsc.VectorSubcoreMesh(core_axis_name="c", subcore_axis_name="s")
    run = pl.kernel(
        _emb_body,
        out_type=jax.ShapeDtypeStruct((NROWS, DIM), jnp.float32),
        mesh=mesh,
        scratch_types=[
            pltpu.VMEM((C,), jnp.int32),
            pltpu.VMEM((C, DIM), jnp.float32),
            pltpu.VMEM((SEQ, DIM), jnp.float32),
            pltpu.SemaphoreType.DMA,
        ],
        compiler_params=pltpu.CompilerParams(use_tc_tiling_on_sc=False),
    )
    out = run(idx, token_table, pos_table)
    return out.reshape(BATCH, SEQ, DIM)
